# Initial kernel scaffold; baseline (speedup 1.0000x reference)
import jax
import jax.numpy as jnp
from jax.experimental import pallas as pl


def kernel(x_0, edge_index_0, batch_0, x_1, edge_index_1, batch_1, W1, b1, W2, b2, W3, b3, W4, b4, Wg0a, bg0a, Wg0b, bg0b, gamma0, beta0, Wg1a, bg1a, Wg1b, bg1b, gamma1, beta1, Wf0a, bf0a, Wf0b, bf0b, Wf1a, bf1a, Wf1b, bf1b):
    raise NotImplementedError("write your pallas kernel here")


# speedup vs baseline: 2.2762x; 2.2762x over previous
import functools

import jax
import jax.numpy as jnp
from jax import lax
from jax.experimental import pallas as pl
from jax.experimental.pallas import tpu as pltpu
from jax.experimental.pallas import tpu_sc as plsc

N0 = 50000
E0 = 800000
G = 512
NC = 2
NS = 16
L = 16
NW = NC * NS
NP = 53248
EB = 200
EP = NW * 128 * EB
GP = 768

def _mesh():
    return plsc.VectorSubcoreMesh(
        core_axis_name="c", subcore_axis_name="s",
        num_cores=NC, num_subcores=NS)


def _make_edge_agg(P, Dc):
    ZR = 128
    RZ = NP // NS // ZR
    RD = NP // NS

    @functools.partial(
        pl.kernel,
        out_type=jax.ShapeDtypeStruct((NC, P, NP, Dc), jnp.float32),
        mesh=_mesh(),
        compiler_params=pltpu.CompilerParams(use_tc_tiling_on_sc=False, needs_layout_passes=False),
        scratch_types=[
            pltpu.VMEM((EB, 128), jnp.int32),
            pltpu.VMEM((EB, 128), jnp.int32),
            pltpu.VMEM((128, Dc), jnp.float32),
            pltpu.VMEM((ZR, Dc), jnp.float32),
            pltpu.VMEM_SHARED((NP, Dc), jnp.float32),
            pltpu.SemaphoreType.DMA,
        ],
    )
    def k(*refs):
        src_hbm, dst_hbm = refs[0], refs[1]
        xparts = refs[2:2 + P]
        out = refs[2 + P]
        src_v, dst_v, rows_v, zbuf, acc, sem = refs[3 + P:]

        cid = lax.axis_index("c")
        sid = lax.axis_index("s")
        wid = sid * NC + cid

        zv = jnp.zeros((L,), jnp.float32)
        for r in range(ZR):
            for c2 in range(Dc // L):
                zbuf[r, pl.ds(c2 * L, L)] = zv

        pltpu.sync_copy(src_hbm.at[pl.ds(wid * EB, EB)], src_v)
        pltpu.sync_copy(dst_hbm.at[pl.ds(wid * EB, EB)], dst_v)

        for p in range(P):
            for i in range(RZ):
                pltpu.sync_copy(zbuf, acc.at[pl.ds(sid * RD + i * ZR, ZR)])
            plsc.subcore_barrier()

            xp = xparts[p]

            def ebody(j, carry):
                cp = pltpu.async_copy(xp.at[src_v.at[j]], rows_v, sem)
                cp.wait()
                pltpu.sync_copy(rows_v, acc.at[dst_v.at[j]], add=True)
                return carry

            lax.fori_loop(0, EB, ebody, 0)
            plsc.subcore_barrier()

            pltpu.sync_copy(acc.at[pl.ds(sid * RD, RD)],
                            out.at[cid, p, pl.ds(sid * RD, RD)])
            plsc.subcore_barrier()

    return k


def _make_gin_mm(P, Dc, Dout, BR=1024):
    NIN = P * (NC + 1)

    def body(*refs):
        ins = refs[:NIN]
        wref, bref, oref = refs[NIN], refs[NIN + 1], refs[NIN + 2]
        acc = jnp.zeros((BR, Dout), jnp.float32)
        for p in range(P):
            wslice = wref[pl.ds(p * Dc, Dc), :]
            part = ins[p][...]
            for c in range(NC):
                part = part + ins[P + c * P + p][...]
            acc = acc + jnp.dot(part, wslice,
                                preferred_element_type=jnp.float32)
        oref[...] = jnp.maximum(acc + bref[...], 0.0)

    grid = (NP // BR,)
    chunk_spec = pl.BlockSpec((BR, Dc), lambda i: (i, 0))
    f = pl.pallas_call(
        body,
        grid=grid,
        in_specs=[chunk_spec] * NIN + [
            pl.BlockSpec((P * Dc, Dout), lambda i: (0, 0)),
            pl.BlockSpec((1, Dout), lambda i: (0, 0)),
        ],
        out_specs=pl.BlockSpec((BR, Dout), lambda i: (i, 0)),
        out_shape=jax.ShapeDtypeStruct((NP, Dout), jnp.float32),
    )
    return f


def _make_pool(Dhp):
    RT = NP // NW
    NB = RT // L
    NV = Dhp // L
    ZSR = 16
    RZ = GP // NS

    @functools.partial(
        pl.kernel,
        out_type=(
            jax.ShapeDtypeStruct((NC, G, Dhp), jnp.float32),
            jax.ShapeDtypeStruct((NC, G + 16, Dhp), jnp.float32),
            jax.ShapeDtypeStruct((NW * 8, Dhp), jnp.float32),
            jax.ShapeDtypeStruct((NW * 8, L), jnp.int32),
        ),
        mesh=_mesh(),
        compiler_params=pltpu.CompilerParams(use_tc_tiling_on_sc=False, needs_layout_passes=False),
        scratch_types=[
            pltpu.VMEM((L, Dhp), jnp.float32),
            pltpu.VMEM((NB, L), jnp.int32),
            pltpu.VMEM((1, Dhp), jnp.float32),
            pltpu.VMEM((8, Dhp), jnp.float32),
            pltpu.VMEM((8, L), jnp.int32),
            pltpu.VMEM((ZSR, Dhp), jnp.float32),
            pltpu.VMEM_SHARED((GP, Dhp), jnp.float32),
            pltpu.SemaphoreType.DMA,
        ],
    )
    def k(h_hbm, b2d_hbm, sumout, maxout, bvout, bidout,
          hblk, bidx, stage, bnd, idst, zst, sumacc, sem):
        cid = lax.axis_index("c")
        sid = lax.axis_index("s")
        wid = sid * NC + cid
        r0 = wid * RT
        lane = lax.iota(jnp.int32, L)

        zv = jnp.zeros((L,), jnp.float32)
        for r in range(ZSR):
            for c2 in range(NV):
                zst[r, pl.ds(c2 * L, L)] = zv

        pltpu.sync_copy(b2d_hbm.at[pl.ds(wid * NB, NB)], bidx)

        for i in range(3):
            pltpu.sync_copy(zst, sumacc.at[pl.ds(sid * RZ + i * ZSR, ZSR)])
        for i in range(3):
            pltpu.sync_copy(zst.at[pl.ds(0, 11)],
                            maxout.at[cid, pl.ds(sid * 33 + i * 11, 11)])
        for r in range(8):
            for c2 in range(NV):
                bnd[r, pl.ds(c2 * L, L)] = zv
        plsc.subcore_barrier()

        bv0 = bidx[0, pl.ds(0, L)]
        s_init = jnp.sum(jnp.where(lane == 0, bv0, 0))

        def flush(s_cur, flushed, m):
            def to_acc():
                for c2 in range(NV):
                    stage[0, pl.ds(c2 * L, L)] = m[c2]
                pltpu.sync_copy(stage, maxout.at[cid, pl.ds(s_cur, 1)])

            def to_slot0():
                for c2 in range(NV):
                    bnd[0, pl.ds(c2 * L, L)] = m[c2]

            lax.cond(flushed > 0, to_acc, to_slot0)

        def blk(i, carry):
            pltpu.sync_copy(h_hbm.at[pl.ds(r0 + i * L, L)], hblk)
            pltpu.sync_copy(hblk, sumacc.at[bidx.at[i]], add=True)
            bv = bidx[i, pl.ds(0, L)]

            def row(j, rcarry):
                s_cur, flushed, id0 = rcarry[0], rcarry[1], rcarry[2]
                m = list(rcarry[3:])
                s_j = jnp.sum(jnp.where(lane == j, bv, 0))
                neq = s_j != s_cur
                lax.cond(neq, lambda: flush(s_cur, flushed, m), lambda: None)
                id0 = jnp.where(neq & (flushed == 0), s_cur, id0)
                flushed = jnp.where(neq, 1, flushed)
                newm = []
                for c2 in range(NV):
                    rv = hblk[j, pl.ds(c2 * L, L)]
                    mv = jnp.where(neq, 0.0, m[c2])
                    newm.append(jnp.maximum(mv, rv))
                return (s_j, flushed, id0) + tuple(newm)

            return lax.fori_loop(0, L, row, carry)

        init = (s_init, jnp.int32(0), jnp.int32(1023)) + tuple(
            jnp.zeros((L,), jnp.float32) for _ in range(NV))
        fin = lax.fori_loop(0, NB, blk, init)
        s_cur, id0 = fin[0], fin[2]
        mfin = list(fin[3:])

        for c2 in range(NV):
            bnd[1, pl.ds(c2 * L, L)] = mfin[c2]
        pltpu.sync_copy(bnd, bvout.at[pl.ds(wid * 8, 8)])

        ids = jnp.where(lane == 0, id0,
                        jnp.where(lane == 1, s_cur, 1023))
        idst[0, pl.ds(0, L)] = ids
        pltpu.sync_copy(idst, bidout.at[pl.ds(wid * 8, 8)])

        plsc.subcore_barrier()

        DR = G // NS
        pltpu.sync_copy(sumacc.at[pl.ds(sid * DR, DR)],
                        sumout.at[cid, pl.ds(sid * DR, DR)])

    return k


def _make_final(Dhp):
    def body(bid_ref, sum_ref, max_ref, bval_ref, batch_ref,
             wa_ref, ba_ref, wb_ref, bb_ref, gam_ref, bet_ref,
             wfa_ref, bfa_ref, wfb_ref, bfb_ref,
             g_out, z_out, mxs):
        segid = lax.broadcasted_iota(jnp.int32, (G, 512), 0)

        def cbody(i, acc):
            chunk = jnp.reshape(batch_ref[pl.ds(i * 512, 512)], (1, 512))
            eq = (chunk == segid).astype(jnp.float32)
            return acc + jnp.sum(eq, axis=1, keepdims=True)

        cnt = lax.fori_loop(0, NP // 512, cbody,
                            jnp.zeros((G, 1), jnp.float32))

        mxs[...] = jnp.maximum(max_ref[0], max_ref[1])

        def mbody(w, carry):
            idv = bid_ref[w]
            valid = idv < G
            iw = jnp.where(valid, idv, 0)
            cur = mxs[pl.ds(iw, 1), :]
            row = bval_ref[pl.ds(w, 1), :]
            mxs[pl.ds(iw, 1), :] = jnp.where(valid,
                                             jnp.maximum(cur, row), cur)
            return carry

        lax.fori_loop(0, 2 * NW, mbody, 0)

        mean = (sum_ref[0] + sum_ref[1]) / jnp.maximum(cnt, 1.0)
        mx = mxs[...]

        t1 = jnp.dot(mean, wa_ref[pl.ds(0, Dhp), :],
                     preferred_element_type=jnp.float32)
        t1 = t1 + jnp.dot(mx, wa_ref[pl.ds(Dhp, Dhp), :],
                          preferred_element_type=jnp.float32)
        t1 = jnp.maximum(t1 + ba_ref[...], 0.0)
        g = jnp.dot(t1, wb_ref[...],
                    preferred_element_type=jnp.float32) + bb_ref[...]
        mu = jnp.mean(g, axis=0, keepdims=True)
        d = g - mu
        v = jnp.mean(d * d, axis=0, keepdims=True)
        gn = gam_ref[...] * d / jnp.sqrt(v + 1e-5) + bet_ref[...]
        g_out[...] = gn
        t2 = jnp.maximum(
            jnp.dot(gn, wfa_ref[...],
                    preferred_element_type=jnp.float32) + bfa_ref[...], 0.0)
        z_out[...] = jnp.dot(t2, wfb_ref[...],
                             preferred_element_type=jnp.float32) + bfb_ref[...]

    f = pl.pallas_call(
        body,
        in_specs=[
            pl.BlockSpec(memory_space=pltpu.SMEM),
        ] + [pl.BlockSpec()] * 14,
        out_specs=[pl.BlockSpec(), pl.BlockSpec()],
        out_shape=(
            jax.ShapeDtypeStruct((G, 512), jnp.float32),
            jax.ShapeDtypeStruct((G, 128), jnp.float32),
        ),
        scratch_shapes=[pltpu.VMEM((G, Dhp), jnp.float32)],
    )
    return f


_edge_agg = {}
_gin_mm = {}
_pool = {}
_final = {}


def _get_edge_agg(P, Dc):
    if (P, Dc) not in _edge_agg:
        _edge_agg[(P, Dc)] = _make_edge_agg(P, Dc)
    return _edge_agg[(P, Dc)]


def _branch(x, src2d, dst2d, batch2d, batchp, W1p, b1p, W2p, b2p,
            P, Dc, Dhp, wa, ba, wb, bb, gam, bet, wfa, bfa, wfb, bfb):
    D = P * Dc
    agg_fn = _get_edge_agg(P, Dc)
    mm1 = _gin_mm.setdefault((P, Dc, D), _make_gin_mm(P, Dc, D))
    mm2 = _gin_mm.setdefault((P, Dc, Dhp), _make_gin_mm(P, Dc, Dhp))
    pool_fn = _pool.setdefault(Dhp, _make_pool(Dhp))
    fin_fn = _final.setdefault(Dhp, _make_final(Dhp))

    xc = [x[:, p * Dc:(p + 1) * Dc] for p in range(P)]
    agg1 = agg_fn(src2d, dst2d, *xc)
    parts1 = [agg1[c, p] for c in range(NC) for p in range(P)]
    h1 = mm1(*xc, *parts1, W1p, b1p)

    h1c = [h1[:, p * Dc:(p + 1) * Dc] for p in range(P)]
    agg2 = agg_fn(src2d, dst2d, *h1c)
    parts2 = [agg2[c, p] for c in range(NC) for p in range(P)]
    h2 = mm2(*h1c, *parts2, W2p, b2p)

    sums, maxs, bval, bid = pool_fn(h2, batch2d)
    maxs = maxs[:, :G]
    bval2 = bval.reshape(NW, 8, Dhp)[:, :2].reshape(2 * NW, Dhp)
    bid2 = bid.reshape(NW, 8, L)[:, 0, :2].reshape(2 * NW)
    g, z = fin_fn(bid2, sums, maxs, bval2, batchp,
                  wa, ba, wb, bb, gam, bet, wfa, bfa, wfb, bfb)
    return g, z[:, :2]


def _pad2(w, r, c):
    return jnp.pad(w, ((0, r - w.shape[0]), (0, c - w.shape[1])))


def _cat_weight(w, dh, dhp, dout):
    z = jnp.zeros((dhp - dh, dout), jnp.float32)
    return jnp.concatenate([w[:dh], z, w[dh:], z], axis=0)


def kernel(x_0, edge_index_0, batch_0, x_1, edge_index_1, batch_1,
           W1, b1, W2, b2, W3, b3, W4, b4,
           Wg0a, bg0a, Wg0b, bg0b, gamma0, beta0,
           Wg1a, bg1a, Wg1b, bg1b, gamma1, beta1,
           Wf0a, bf0a, Wf0b, bf0b, Wf1a, bf1a, Wf1b, bf1b):
    x0p = jnp.pad(x_0, ((0, NP - N0), (0, 96 - 93)))
    x1p = jnp.pad(x_1, ((0, NP - N0), (0, 48 - 43)))
    src0 = jnp.pad(edge_index_0[0], (0, EP - E0)).reshape(EP // 128, 128)
    dst0 = jnp.pad(edge_index_0[1], (0, EP - E0),
                   constant_values=N0).reshape(EP // 128, 128)
    src1 = jnp.pad(edge_index_1[0], (0, EP - E0)).reshape(EP // 128, 128)
    dst1 = jnp.pad(edge_index_1[1], (0, EP - E0),
                   constant_values=N0).reshape(EP // 128, 128)
    b0p = jnp.pad(batch_0, (0, NP - N0), constant_values=G)
    b1p_ = jnp.pad(batch_1, (0, NP - N0), constant_values=G)
    b0_2d = b0p.reshape(NP // L, L)
    b1_2d = b1p_.reshape(NP // L, L)

    W1p = _pad2(W1, 96, 96)
    W2p = _pad2(W2, 96, 960)
    W3p = _pad2(W3, 48, 48)
    W4p = _pad2(W4, 48, 448)
    b1v = _pad2(b1[None, :], 1, 96)
    b2v = _pad2(b2[None, :], 1, 960)
    b3v = _pad2(b3[None, :], 1, 48)
    b4v = _pad2(b4[None, :], 1, 448)

    wg0 = _cat_weight(Wg0a, 930, 960, 1024)
    wg1 = _cat_weight(Wg1a, 430, 448, 1024)
    wf0b = _pad2(Wf0b, 256, 128)
    wf1b = _pad2(Wf1b, 256, 128)
    bf0bv = _pad2(bf0b[None, :], 1, 128)
    bf1bv = _pad2(bf1b[None, :], 1, 128)

    g0, z0 = _branch(x0p, src0, dst0, b0_2d, b0p, W1p, b1v, W2p, b2v,
                     6, 16, 960,
                     wg0, bg0a[None, :], Wg0b, bg0b[None, :],
                     gamma0[None, :], beta0[None, :],
                     Wf0a, bf0a[None, :], wf0b, bf0bv)
    g1, z1 = _branch(x1p, src1, dst1, b1_2d, b1p_, W3p, b3v, W4p, b4v,
                     3, 16, 448,
                     wg1, bg1a[None, :], Wg1b, bg1b[None, :],
                     gamma1[None, :], beta1[None, :],
                     Wf1a, bf1a[None, :], wf1b, bf1bv)
    return (z0, g0, g1, z1)


# speedup vs baseline: 2.6592x; 1.1683x over previous
import functools

import jax
import jax.numpy as jnp
from jax import lax
from jax.experimental import pallas as pl
from jax.experimental.pallas import tpu as pltpu
from jax.experimental.pallas import tpu_sc as plsc

N0 = 50000
E0 = 800000
G = 512
NC = 2
NS = 16
L = 16
NW = NC * NS
NP = 53248
EB = 200
EP = NW * 128 * EB
GP = 768

def _mesh():
    return plsc.VectorSubcoreMesh(
        core_axis_name="c", subcore_axis_name="s",
        num_cores=NC, num_subcores=NS)


def _make_edge_agg(P, Dc):
    ZR = 128
    RZ = NP // NS // ZR
    RD = NP // NS

    @functools.partial(
        pl.kernel,
        out_type=jax.ShapeDtypeStruct((NC, P, NP, Dc), jnp.float32),
        mesh=_mesh(),
        compiler_params=pltpu.CompilerParams(use_tc_tiling_on_sc=False, needs_layout_passes=False),
        scratch_types=[
            pltpu.VMEM((EB, 128), jnp.int32),
            pltpu.VMEM((EB, 128), jnp.int32),
            pltpu.VMEM((128, Dc), jnp.float32),
            pltpu.VMEM((128, Dc), jnp.float32),
            pltpu.VMEM((ZR, Dc), jnp.float32),
            pltpu.VMEM_SHARED((NP, Dc), jnp.float32),
            pltpu.SemaphoreType.DMA,
            pltpu.SemaphoreType.DMA,
            pltpu.SemaphoreType.DMA,
            pltpu.SemaphoreType.DMA,
        ],
    )
    def k(*refs):
        src_hbm, dst_hbm = refs[0], refs[1]
        xparts = refs[2:2 + P]
        out = refs[2 + P]
        (src_v, dst_v, rows0, rows1, zbuf, acc,
         semg0, semg1, sems0, sems1) = refs[3 + P:]

        cid = lax.axis_index("c")
        sid = lax.axis_index("s")
        wid = sid * NC + cid

        zv = jnp.zeros((L,), jnp.float32)
        for r in range(ZR):
            for c2 in range(Dc // L):
                zbuf[r, pl.ds(c2 * L, L)] = zv

        pltpu.sync_copy(src_hbm.at[pl.ds(wid * EB, EB)], src_v)
        pltpu.sync_copy(dst_hbm.at[pl.ds(wid * EB, EB)], dst_v)

        for p in range(P):
            for i in range(RZ):
                pltpu.sync_copy(zbuf, acc.at[pl.ds(sid * RD + i * ZR, ZR)])
            plsc.subcore_barrier()

            xp = xparts[p]
            dummy = xp.at[pl.ds(0, 128)]

            pltpu.async_copy(xp.at[src_v.at[0]], rows0, semg0)
            pltpu.async_copy(xp.at[src_v.at[1]], rows1, semg1)

            def ebody(i, carry):
                j0 = 2 * i
                j1 = 2 * i + 1
                jn0 = jnp.minimum(j0 + 2, EB - 1)
                jn1 = jnp.minimum(j1 + 2, EB - 1)
                pltpu.make_async_copy(dummy, rows0, semg0).wait()
                pltpu.async_copy(rows0, acc.at[dst_v.at[j0]], sems0,
                                 add=True)
                pltpu.make_async_copy(dummy, rows1, semg1).wait()
                pltpu.async_copy(rows1, acc.at[dst_v.at[j1]], sems1,
                                 add=True)
                pltpu.make_async_copy(rows0, acc.at[dst_v.at[j0]],
                                      sems0).wait()
                pltpu.async_copy(xp.at[src_v.at[jn0]], rows0, semg0)
                pltpu.make_async_copy(rows1, acc.at[dst_v.at[j1]],
                                      sems1).wait()
                pltpu.async_copy(xp.at[src_v.at[jn1]], rows1, semg1)
                return carry

            lax.fori_loop(0, EB // 2, ebody, 0)
            pltpu.make_async_copy(dummy, rows0, semg0).wait()
            pltpu.make_async_copy(dummy, rows1, semg1).wait()
            plsc.subcore_barrier()

            pltpu.sync_copy(acc.at[pl.ds(sid * RD, RD)],
                            out.at[cid, p, pl.ds(sid * RD, RD)])
            plsc.subcore_barrier()

    return k


def _make_gin_mm(P, Dc, Dout, BR=1024):
    NIN = P * (NC + 1)

    def body(*refs):
        ins = refs[:NIN]
        wref, bref, oref = refs[NIN], refs[NIN + 1], refs[NIN + 2]
        acc = jnp.zeros((BR, Dout), jnp.float32)
        for p in range(P):
            wslice = wref[pl.ds(p * Dc, Dc), :]
            part = ins[p][...]
            for c in range(NC):
                part = part + ins[P + c * P + p][...]
            acc = acc + jnp.dot(part, wslice,
                                preferred_element_type=jnp.float32)
        oref[...] = jnp.maximum(acc + bref[...], 0.0)

    grid = (NP // BR,)
    chunk_spec = pl.BlockSpec((BR, Dc), lambda i: (i, 0))
    f = pl.pallas_call(
        body,
        grid=grid,
        in_specs=[chunk_spec] * NIN + [
            pl.BlockSpec((P * Dc, Dout), lambda i: (0, 0)),
            pl.BlockSpec((1, Dout), lambda i: (0, 0)),
        ],
        out_specs=pl.BlockSpec((BR, Dout), lambda i: (i, 0)),
        out_shape=jax.ShapeDtypeStruct((NP, Dout), jnp.float32),
    )
    return f


def _make_pool(Dhp):
    RT = NP // NW
    NB = RT // L
    NV = Dhp // L
    ZSR = 16
    RZ = GP // NS

    @functools.partial(
        pl.kernel,
        out_type=(
            jax.ShapeDtypeStruct((NC, G, Dhp), jnp.float32),
            jax.ShapeDtypeStruct((NC, G + 16, Dhp), jnp.float32),
            jax.ShapeDtypeStruct((NW * 8, Dhp), jnp.float32),
            jax.ShapeDtypeStruct((NW * 8, L), jnp.int32),
        ),
        mesh=_mesh(),
        compiler_params=pltpu.CompilerParams(use_tc_tiling_on_sc=False, needs_layout_passes=False),
        scratch_types=[
            pltpu.VMEM((L, Dhp), jnp.float32),
            pltpu.VMEM((NB, L), jnp.int32),
            pltpu.VMEM((1, Dhp), jnp.float32),
            pltpu.VMEM((8, Dhp), jnp.float32),
            pltpu.VMEM((8, L), jnp.int32),
            pltpu.VMEM((ZSR, Dhp), jnp.float32),
            pltpu.VMEM_SHARED((GP, Dhp), jnp.float32),
            pltpu.SemaphoreType.DMA,
        ],
    )
    def k(h_hbm, b2d_hbm, sumout, maxout, bvout, bidout,
          hblk, bidx, stage, bnd, idst, zst, sumacc, sem):
        cid = lax.axis_index("c")
        sid = lax.axis_index("s")
        wid = sid * NC + cid
        r0 = wid * RT
        lane = lax.iota(jnp.int32, L)

        zv = jnp.zeros((L,), jnp.float32)
        for r in range(ZSR):
            for c2 in range(NV):
                zst[r, pl.ds(c2 * L, L)] = zv

        pltpu.sync_copy(b2d_hbm.at[pl.ds(wid * NB, NB)], bidx)

        for i in range(3):
            pltpu.sync_copy(zst, sumacc.at[pl.ds(sid * RZ + i * ZSR, ZSR)])
        for i in range(3):
            pltpu.sync_copy(zst.at[pl.ds(0, 11)],
                            maxout.at[cid, pl.ds(sid * 33 + i * 11, 11)])
        for r in range(8):
            for c2 in range(NV):
                bnd[r, pl.ds(c2 * L, L)] = zv
        plsc.subcore_barrier()

        bv0 = bidx[0, pl.ds(0, L)]
        s_init = jnp.sum(jnp.where(lane == 0, bv0, 0))

        def flush(s_cur, flushed, m):
            def to_acc():
                for c2 in range(NV):
                    stage[0, pl.ds(c2 * L, L)] = m[c2]
                pltpu.sync_copy(stage, maxout.at[cid, pl.ds(s_cur, 1)])

            def to_slot0():
                for c2 in range(NV):
                    bnd[0, pl.ds(c2 * L, L)] = m[c2]

            lax.cond(flushed > 0, to_acc, to_slot0)

        def blk(i, carry):
            pltpu.sync_copy(h_hbm.at[pl.ds(r0 + i * L, L)], hblk)
            pltpu.sync_copy(hblk, sumacc.at[bidx.at[i]], add=True)
            bv = bidx[i, pl.ds(0, L)]

            def row(j, rcarry):
                s_cur, flushed, id0 = rcarry[0], rcarry[1], rcarry[2]
                m = list(rcarry[3:])
                s_j = jnp.sum(jnp.where(lane == j, bv, 0))
                neq = s_j != s_cur
                lax.cond(neq, lambda: flush(s_cur, flushed, m), lambda: None)
                id0 = jnp.where(neq & (flushed == 0), s_cur, id0)
                flushed = jnp.where(neq, 1, flushed)
                newm = []
                for c2 in range(NV):
                    rv = hblk[j, pl.ds(c2 * L, L)]
                    mv = jnp.where(neq, 0.0, m[c2])
                    newm.append(jnp.maximum(mv, rv))
                return (s_j, flushed, id0) + tuple(newm)

            return lax.fori_loop(0, L, row, carry)

        init = (s_init, jnp.int32(0), jnp.int32(1023)) + tuple(
            jnp.zeros((L,), jnp.float32) for _ in range(NV))
        fin = lax.fori_loop(0, NB, blk, init)
        s_cur, id0 = fin[0], fin[2]
        mfin = list(fin[3:])

        for c2 in range(NV):
            bnd[1, pl.ds(c2 * L, L)] = mfin[c2]
        pltpu.sync_copy(bnd, bvout.at[pl.ds(wid * 8, 8)])

        ids = jnp.where(lane == 0, id0,
                        jnp.where(lane == 1, s_cur, 1023))
        idst[0, pl.ds(0, L)] = ids
        pltpu.sync_copy(idst, bidout.at[pl.ds(wid * 8, 8)])

        plsc.subcore_barrier()

        DR = G // NS
        pltpu.sync_copy(sumacc.at[pl.ds(sid * DR, DR)],
                        sumout.at[cid, pl.ds(sid * DR, DR)])

    return k


def _make_final(Dhp):
    def body(bid_ref, sum_ref, max_ref, bval_ref, batch_ref,
             wa_ref, ba_ref, wb_ref, bb_ref, gam_ref, bet_ref,
             wfa_ref, bfa_ref, wfb_ref, bfb_ref,
             g_out, z_out, mxs):
        segid = lax.broadcasted_iota(jnp.int32, (G, 512), 0)

        def cbody(i, acc):
            chunk = jnp.reshape(batch_ref[pl.ds(i * 512, 512)], (1, 512))
            eq = (chunk == segid).astype(jnp.float32)
            return acc + jnp.sum(eq, axis=1, keepdims=True)

        cnt = lax.fori_loop(0, NP // 512, cbody,
                            jnp.zeros((G, 1), jnp.float32))

        mxs[...] = jnp.maximum(max_ref[0], max_ref[1])

        def mbody(w, carry):
            idv = bid_ref[w]
            valid = idv < G
            iw = jnp.where(valid, idv, 0)
            cur = mxs[pl.ds(iw, 1), :]
            row = bval_ref[pl.ds(w, 1), :]
            mxs[pl.ds(iw, 1), :] = jnp.where(valid,
                                             jnp.maximum(cur, row), cur)
            return carry

        lax.fori_loop(0, 2 * NW, mbody, 0)

        mean = (sum_ref[0] + sum_ref[1]) / jnp.maximum(cnt, 1.0)
        mx = mxs[...]

        t1 = jnp.dot(mean, wa_ref[pl.ds(0, Dhp), :],
                     preferred_element_type=jnp.float32)
        t1 = t1 + jnp.dot(mx, wa_ref[pl.ds(Dhp, Dhp), :],
                          preferred_element_type=jnp.float32)
        t1 = jnp.maximum(t1 + ba_ref[...], 0.0)
        g = jnp.dot(t1, wb_ref[...],
                    preferred_element_type=jnp.float32) + bb_ref[...]
        mu = jnp.mean(g, axis=0, keepdims=True)
        d = g - mu
        v = jnp.mean(d * d, axis=0, keepdims=True)
        gn = gam_ref[...] * d / jnp.sqrt(v + 1e-5) + bet_ref[...]
        g_out[...] = gn
        t2 = jnp.maximum(
            jnp.dot(gn, wfa_ref[...],
                    preferred_element_type=jnp.float32) + bfa_ref[...], 0.0)
        z_out[...] = jnp.dot(t2, wfb_ref[...],
                             preferred_element_type=jnp.float32) + bfb_ref[...]

    f = pl.pallas_call(
        body,
        in_specs=[
            pl.BlockSpec(memory_space=pltpu.SMEM),
        ] + [pl.BlockSpec()] * 14,
        out_specs=[pl.BlockSpec(), pl.BlockSpec()],
        out_shape=(
            jax.ShapeDtypeStruct((G, 512), jnp.float32),
            jax.ShapeDtypeStruct((G, 128), jnp.float32),
        ),
        scratch_shapes=[pltpu.VMEM((G, Dhp), jnp.float32)],
    )
    return f


_edge_agg = {}
_gin_mm = {}
_pool = {}
_final = {}


def _get_edge_agg(P, Dc):
    if (P, Dc) not in _edge_agg:
        _edge_agg[(P, Dc)] = _make_edge_agg(P, Dc)
    return _edge_agg[(P, Dc)]


def _branch(x, src2d, dst2d, batch2d, batchp, W1p, b1p, W2p, b2p,
            P, Dc, Dhp, wa, ba, wb, bb, gam, bet, wfa, bfa, wfb, bfb):
    D = P * Dc
    agg_fn = _get_edge_agg(P, Dc)
    mm1 = _gin_mm.setdefault((P, Dc, D), _make_gin_mm(P, Dc, D))
    mm2 = _gin_mm.setdefault((P, Dc, Dhp), _make_gin_mm(P, Dc, Dhp))
    pool_fn = _pool.setdefault(Dhp, _make_pool(Dhp))
    fin_fn = _final.setdefault(Dhp, _make_final(Dhp))

    xc = [x[:, p * Dc:(p + 1) * Dc] for p in range(P)]
    agg1 = agg_fn(src2d, dst2d, *xc)
    parts1 = [agg1[c, p] for c in range(NC) for p in range(P)]
    h1 = mm1(*xc, *parts1, W1p, b1p)

    h1c = [h1[:, p * Dc:(p + 1) * Dc] for p in range(P)]
    agg2 = agg_fn(src2d, dst2d, *h1c)
    parts2 = [agg2[c, p] for c in range(NC) for p in range(P)]
    h2 = mm2(*h1c, *parts2, W2p, b2p)

    sums, maxs, bval, bid = pool_fn(h2, batch2d)
    maxs = maxs[:, :G]
    bval2 = bval.reshape(NW, 8, Dhp)[:, :2].reshape(2 * NW, Dhp)
    bid2 = bid.reshape(NW, 8, L)[:, 0, :2].reshape(2 * NW)
    g, z = fin_fn(bid2, sums, maxs, bval2, batchp,
                  wa, ba, wb, bb, gam, bet, wfa, bfa, wfb, bfb)
    return g, z[:, :2]


def _pad2(w, r, c):
    return jnp.pad(w, ((0, r - w.shape[0]), (0, c - w.shape[1])))


def _cat_weight(w, dh, dhp, dout):
    z = jnp.zeros((dhp - dh, dout), jnp.float32)
    return jnp.concatenate([w[:dh], z, w[dh:], z], axis=0)


def kernel(x_0, edge_index_0, batch_0, x_1, edge_index_1, batch_1,
           W1, b1, W2, b2, W3, b3, W4, b4,
           Wg0a, bg0a, Wg0b, bg0b, gamma0, beta0,
           Wg1a, bg1a, Wg1b, bg1b, gamma1, beta1,
           Wf0a, bf0a, Wf0b, bf0b, Wf1a, bf1a, Wf1b, bf1b):
    x0p = jnp.pad(x_0, ((0, NP - N0), (0, 96 - 93)))
    x1p = jnp.pad(x_1, ((0, NP - N0), (0, 48 - 43)))
    src0 = jnp.pad(edge_index_0[0], (0, EP - E0)).reshape(EP // 128, 128)
    dst0 = jnp.pad(edge_index_0[1], (0, EP - E0),
                   constant_values=N0).reshape(EP // 128, 128)
    src1 = jnp.pad(edge_index_1[0], (0, EP - E0)).reshape(EP // 128, 128)
    dst1 = jnp.pad(edge_index_1[1], (0, EP - E0),
                   constant_values=N0).reshape(EP // 128, 128)
    b0p = jnp.pad(batch_0, (0, NP - N0), constant_values=G)
    b1p_ = jnp.pad(batch_1, (0, NP - N0), constant_values=G)
    b0_2d = b0p.reshape(NP // L, L)
    b1_2d = b1p_.reshape(NP // L, L)

    W1p = _pad2(W1, 96, 96)
    W2p = _pad2(W2, 96, 960)
    W3p = _pad2(W3, 48, 48)
    W4p = _pad2(W4, 48, 448)
    b1v = _pad2(b1[None, :], 1, 96)
    b2v = _pad2(b2[None, :], 1, 960)
    b3v = _pad2(b3[None, :], 1, 48)
    b4v = _pad2(b4[None, :], 1, 448)

    wg0 = _cat_weight(Wg0a, 930, 960, 1024)
    wg1 = _cat_weight(Wg1a, 430, 448, 1024)
    wf0b = _pad2(Wf0b, 256, 128)
    wf1b = _pad2(Wf1b, 256, 128)
    bf0bv = _pad2(bf0b[None, :], 1, 128)
    bf1bv = _pad2(bf1b[None, :], 1, 128)

    g0, z0 = _branch(x0p, src0, dst0, b0_2d, b0p, W1p, b1v, W2p, b2v,
                     6, 16, 960,
                     wg0, bg0a[None, :], Wg0b, bg0b[None, :],
                     gamma0[None, :], beta0[None, :],
                     Wf0a, bf0a[None, :], wf0b, bf0bv)
    g1, z1 = _branch(x1p, src1, dst1, b1_2d, b1p_, W3p, b3v, W4p, b4v,
                     3, 16, 448,
                     wg1, bg1a[None, :], Wg1b, bg1b[None, :],
                     gamma1[None, :], beta1[None, :],
                     Wf1a, bf1a[None, :], wf1b, bf1bv)
    return (z0, g0, g1, z1)


# speedup vs baseline: 2.7862x; 1.0477x over previous
import functools

import jax
import jax.numpy as jnp
from jax import lax
from jax.experimental import pallas as pl
from jax.experimental.pallas import tpu as pltpu
from jax.experimental.pallas import tpu_sc as plsc

N0 = 50000
E0 = 800000
G = 512
NC = 2
NS = 16
L = 16
NW = NC * NS
NP = 53248
EB = 200
EP = NW * 128 * EB
GP = 768

def _mesh():
    return plsc.VectorSubcoreMesh(
        core_axis_name="c", subcore_axis_name="s",
        num_cores=NC, num_subcores=NS)


def _make_edge_agg(P, Dc):
    ZR = 128
    RZ = NP // NS // ZR
    RD = NP // NS

    @functools.partial(
        pl.kernel,
        out_type=jax.ShapeDtypeStruct((NC, P, NP, Dc), jnp.float32),
        mesh=_mesh(),
        compiler_params=pltpu.CompilerParams(use_tc_tiling_on_sc=False, needs_layout_passes=False),
        scratch_types=[
            pltpu.VMEM((EB, 128), jnp.int32),
            pltpu.VMEM((EB, 128), jnp.int32),
            pltpu.VMEM((128, Dc), jnp.float32),
            pltpu.VMEM((128, Dc), jnp.float32),
            pltpu.VMEM((128, Dc), jnp.float32),
            pltpu.VMEM((128, Dc), jnp.float32),
            pltpu.VMEM((ZR, Dc), jnp.float32),
            pltpu.VMEM_SHARED((NP, Dc), jnp.float32),
            pltpu.SemaphoreType.DMA,
            pltpu.SemaphoreType.DMA,
            pltpu.SemaphoreType.DMA,
            pltpu.SemaphoreType.DMA,
            pltpu.SemaphoreType.DMA,
            pltpu.SemaphoreType.DMA,
            pltpu.SemaphoreType.DMA,
            pltpu.SemaphoreType.DMA,
        ],
    )
    def k(*refs):
        src_hbm, dst_hbm = refs[0], refs[1]
        xparts = refs[2:2 + P]
        out = refs[2 + P]
        scr = refs[3 + P:]
        src_v, dst_v = scr[0], scr[1]
        rows = scr[2:6]
        zbuf, acc = scr[6], scr[7]
        semg = scr[8:12]
        sems = scr[12:16]

        cid = lax.axis_index("c")
        sid = lax.axis_index("s")
        wid = sid * NC + cid

        zv = jnp.zeros((L,), jnp.float32)
        for r in range(ZR):
            for c2 in range(Dc // L):
                zbuf[r, pl.ds(c2 * L, L)] = zv

        pltpu.sync_copy(src_hbm.at[pl.ds(wid * EB, EB)], src_v)
        pltpu.sync_copy(dst_hbm.at[pl.ds(wid * EB, EB)], dst_v)

        for p in range(P):
            for i in range(RZ):
                pltpu.sync_copy(zbuf, acc.at[pl.ds(sid * RD + i * ZR, ZR)])
            plsc.subcore_barrier()

            xp = xparts[p]
            dummy = xp.at[pl.ds(0, 128)]

            for b in range(4):
                pltpu.async_copy(xp.at[src_v.at[b]], rows[b], semg[b])

            def ebody(i, carry):
                for b in range(4):
                    j = 4 * i + b
                    pltpu.make_async_copy(dummy, rows[b], semg[b]).wait()
                    pltpu.async_copy(rows[b], acc.at[dst_v.at[j]], sems[b],
                                     add=True)
                for b in range(4):
                    j = 4 * i + b
                    jn = jnp.minimum(j + 4, EB - 1)
                    pltpu.make_async_copy(rows[b], acc.at[dst_v.at[j]],
                                          sems[b]).wait()
                    pltpu.async_copy(xp.at[src_v.at[jn]], rows[b], semg[b])
                return carry

            lax.fori_loop(0, EB // 4, ebody, 0)
            for b in range(4):
                pltpu.make_async_copy(dummy, rows[b], semg[b]).wait()
            plsc.subcore_barrier()

            pltpu.sync_copy(acc.at[pl.ds(sid * RD, RD)],
                            out.at[cid, p, pl.ds(sid * RD, RD)])
            plsc.subcore_barrier()

    return k


def _make_gin_mm(P, Dc, Dout, BR=1024):
    NIN = P * (NC + 1)

    def body(*refs):
        ins = refs[:NIN]
        wref, bref, oref = refs[NIN], refs[NIN + 1], refs[NIN + 2]
        acc = jnp.zeros((BR, Dout), jnp.float32)
        for p in range(P):
            wslice = wref[pl.ds(p * Dc, Dc), :]
            part = ins[p][...]
            for c in range(NC):
                part = part + ins[P + c * P + p][...]
            acc = acc + jnp.dot(part, wslice,
                                preferred_element_type=jnp.float32)
        oref[...] = jnp.maximum(acc + bref[...], 0.0)

    grid = (NP // BR,)
    chunk_spec = pl.BlockSpec((BR, Dc), lambda i: (i, 0))
    f = pl.pallas_call(
        body,
        grid=grid,
        in_specs=[chunk_spec] * NIN + [
            pl.BlockSpec((P * Dc, Dout), lambda i: (0, 0)),
            pl.BlockSpec((1, Dout), lambda i: (0, 0)),
        ],
        out_specs=pl.BlockSpec((BR, Dout), lambda i: (i, 0)),
        out_shape=jax.ShapeDtypeStruct((NP, Dout), jnp.float32),
    )
    return f


def _make_pool(Dhp):
    RT = NP // NW
    NB = RT // L
    NV = Dhp // L
    ZSR = 16
    RZ = GP // NS

    @functools.partial(
        pl.kernel,
        out_type=(
            jax.ShapeDtypeStruct((NC, G, Dhp), jnp.float32),
            jax.ShapeDtypeStruct((NC, G + 16, Dhp), jnp.float32),
            jax.ShapeDtypeStruct((NW * 8, Dhp), jnp.float32),
            jax.ShapeDtypeStruct((NW * 8, L), jnp.int32),
        ),
        mesh=_mesh(),
        compiler_params=pltpu.CompilerParams(use_tc_tiling_on_sc=False, needs_layout_passes=False),
        scratch_types=[
            pltpu.VMEM((L, Dhp), jnp.float32),
            pltpu.VMEM((NB, L), jnp.int32),
            pltpu.VMEM((1, Dhp), jnp.float32),
            pltpu.VMEM((8, Dhp), jnp.float32),
            pltpu.VMEM((8, L), jnp.int32),
            pltpu.VMEM((ZSR, Dhp), jnp.float32),
            pltpu.VMEM_SHARED((GP, Dhp), jnp.float32),
            pltpu.SemaphoreType.DMA,
        ],
    )
    def k(h_hbm, b2d_hbm, sumout, maxout, bvout, bidout,
          hblk, bidx, stage, bnd, idst, zst, sumacc, sem):
        cid = lax.axis_index("c")
        sid = lax.axis_index("s")
        wid = sid * NC + cid
        r0 = wid * RT
        lane = lax.iota(jnp.int32, L)

        zv = jnp.zeros((L,), jnp.float32)
        for r in range(ZSR):
            for c2 in range(NV):
                zst[r, pl.ds(c2 * L, L)] = zv

        pltpu.sync_copy(b2d_hbm.at[pl.ds(wid * NB, NB)], bidx)

        for i in range(3):
            pltpu.sync_copy(zst, sumacc.at[pl.ds(sid * RZ + i * ZSR, ZSR)])
        for i in range(3):
            pltpu.sync_copy(zst.at[pl.ds(0, 11)],
                            maxout.at[cid, pl.ds(sid * 33 + i * 11, 11)])
        for r in range(8):
            for c2 in range(NV):
                bnd[r, pl.ds(c2 * L, L)] = zv
        plsc.subcore_barrier()

        bv0 = bidx[0, pl.ds(0, L)]
        s_init = jnp.sum(jnp.where(lane == 0, bv0, 0))

        def flush(s_cur, flushed, m):
            def to_acc():
                for c2 in range(NV):
                    stage[0, pl.ds(c2 * L, L)] = m[c2]
                pltpu.sync_copy(stage, maxout.at[cid, pl.ds(s_cur, 1)])

            def to_slot0():
                for c2 in range(NV):
                    bnd[0, pl.ds(c2 * L, L)] = m[c2]

            lax.cond(flushed > 0, to_acc, to_slot0)

        def blk(i, carry):
            pltpu.sync_copy(h_hbm.at[pl.ds(r0 + i * L, L)], hblk)
            pltpu.sync_copy(hblk, sumacc.at[bidx.at[i]], add=True)
            bv = bidx[i, pl.ds(0, L)]

            def row(j, rcarry):
                s_cur, flushed, id0 = rcarry[0], rcarry[1], rcarry[2]
                m = list(rcarry[3:])
                s_j = jnp.sum(jnp.where(lane == j, bv, 0))
                neq = s_j != s_cur
                lax.cond(neq, lambda: flush(s_cur, flushed, m), lambda: None)
                id0 = jnp.where(neq & (flushed == 0), s_cur, id0)
                flushed = jnp.where(neq, 1, flushed)
                newm = []
                for c2 in range(NV):
                    rv = hblk[j, pl.ds(c2 * L, L)]
                    mv = jnp.where(neq, 0.0, m[c2])
                    newm.append(jnp.maximum(mv, rv))
                return (s_j, flushed, id0) + tuple(newm)

            return lax.fori_loop(0, L, row, carry)

        init = (s_init, jnp.int32(0), jnp.int32(1023)) + tuple(
            jnp.zeros((L,), jnp.float32) for _ in range(NV))
        fin = lax.fori_loop(0, NB, blk, init)
        s_cur, id0 = fin[0], fin[2]
        mfin = list(fin[3:])

        for c2 in range(NV):
            bnd[1, pl.ds(c2 * L, L)] = mfin[c2]
        pltpu.sync_copy(bnd, bvout.at[pl.ds(wid * 8, 8)])

        ids = jnp.where(lane == 0, id0,
                        jnp.where(lane == 1, s_cur, 1023))
        idst[0, pl.ds(0, L)] = ids
        pltpu.sync_copy(idst, bidout.at[pl.ds(wid * 8, 8)])

        plsc.subcore_barrier()

        DR = G // NS
        pltpu.sync_copy(sumacc.at[pl.ds(sid * DR, DR)],
                        sumout.at[cid, pl.ds(sid * DR, DR)])

    return k


def _make_final(Dhp):
    def body(bid_ref, sum_ref, max_ref, bval_ref, batch_ref,
             wa_ref, ba_ref, wb_ref, bb_ref, gam_ref, bet_ref,
             wfa_ref, bfa_ref, wfb_ref, bfb_ref,
             g_out, z_out, mxs):
        segid = lax.broadcasted_iota(jnp.int32, (G, 512), 0)

        def cbody(i, acc):
            chunk = jnp.reshape(batch_ref[pl.ds(i * 512, 512)], (1, 512))
            eq = (chunk == segid).astype(jnp.float32)
            return acc + jnp.sum(eq, axis=1, keepdims=True)

        cnt = lax.fori_loop(0, NP // 512, cbody,
                            jnp.zeros((G, 1), jnp.float32))

        mxs[...] = jnp.maximum(max_ref[0], max_ref[1])

        def mbody(w, carry):
            idv = bid_ref[w]
            valid = idv < G
            iw = jnp.where(valid, idv, 0)
            cur = mxs[pl.ds(iw, 1), :]
            row = bval_ref[pl.ds(w, 1), :]
            mxs[pl.ds(iw, 1), :] = jnp.where(valid,
                                             jnp.maximum(cur, row), cur)
            return carry

        lax.fori_loop(0, 2 * NW, mbody, 0)

        mean = (sum_ref[0] + sum_ref[1]) / jnp.maximum(cnt, 1.0)
        mx = mxs[...]

        t1 = jnp.dot(mean, wa_ref[pl.ds(0, Dhp), :],
                     preferred_element_type=jnp.float32)
        t1 = t1 + jnp.dot(mx, wa_ref[pl.ds(Dhp, Dhp), :],
                          preferred_element_type=jnp.float32)
        t1 = jnp.maximum(t1 + ba_ref[...], 0.0)
        g = jnp.dot(t1, wb_ref[...],
                    preferred_element_type=jnp.float32) + bb_ref[...]
        mu = jnp.mean(g, axis=0, keepdims=True)
        d = g - mu
        v = jnp.mean(d * d, axis=0, keepdims=True)
        gn = gam_ref[...] * d / jnp.sqrt(v + 1e-5) + bet_ref[...]
        g_out[...] = gn
        t2 = jnp.maximum(
            jnp.dot(gn, wfa_ref[...],
                    preferred_element_type=jnp.float32) + bfa_ref[...], 0.0)
        z_out[...] = jnp.dot(t2, wfb_ref[...],
                             preferred_element_type=jnp.float32) + bfb_ref[...]

    f = pl.pallas_call(
        body,
        in_specs=[
            pl.BlockSpec(memory_space=pltpu.SMEM),
        ] + [pl.BlockSpec()] * 14,
        out_specs=[pl.BlockSpec(), pl.BlockSpec()],
        out_shape=(
            jax.ShapeDtypeStruct((G, 512), jnp.float32),
            jax.ShapeDtypeStruct((G, 128), jnp.float32),
        ),
        scratch_shapes=[pltpu.VMEM((G, Dhp), jnp.float32)],
    )
    return f


_edge_agg = {}
_gin_mm = {}
_pool = {}
_final = {}


def _get_edge_agg(P, Dc):
    if (P, Dc) not in _edge_agg:
        _edge_agg[(P, Dc)] = _make_edge_agg(P, Dc)
    return _edge_agg[(P, Dc)]


def _branch(x, src2d, dst2d, batch2d, batchp, W1p, b1p, W2p, b2p,
            P, Dc, Dhp, wa, ba, wb, bb, gam, bet, wfa, bfa, wfb, bfb):
    D = P * Dc
    agg_fn = _get_edge_agg(P, Dc)
    mm1 = _gin_mm.setdefault((P, Dc, D), _make_gin_mm(P, Dc, D))
    mm2 = _gin_mm.setdefault((P, Dc, Dhp), _make_gin_mm(P, Dc, Dhp))
    pool_fn = _pool.setdefault(Dhp, _make_pool(Dhp))
    fin_fn = _final.setdefault(Dhp, _make_final(Dhp))

    xc = [x[:, p * Dc:(p + 1) * Dc] for p in range(P)]
    agg1 = agg_fn(src2d, dst2d, *xc)
    parts1 = [agg1[c, p] for c in range(NC) for p in range(P)]
    h1 = mm1(*xc, *parts1, W1p, b1p)

    h1c = [h1[:, p * Dc:(p + 1) * Dc] for p in range(P)]
    agg2 = agg_fn(src2d, dst2d, *h1c)
    parts2 = [agg2[c, p] for c in range(NC) for p in range(P)]
    h2 = mm2(*h1c, *parts2, W2p, b2p)

    sums, maxs, bval, bid = pool_fn(h2, batch2d)
    maxs = maxs[:, :G]
    bval2 = bval.reshape(NW, 8, Dhp)[:, :2].reshape(2 * NW, Dhp)
    bid2 = bid.reshape(NW, 8, L)[:, 0, :2].reshape(2 * NW)
    g, z = fin_fn(bid2, sums, maxs, bval2, batchp,
                  wa, ba, wb, bb, gam, bet, wfa, bfa, wfb, bfb)
    return g, z[:, :2]


def _pad2(w, r, c):
    return jnp.pad(w, ((0, r - w.shape[0]), (0, c - w.shape[1])))


def _cat_weight(w, dh, dhp, dout):
    z = jnp.zeros((dhp - dh, dout), jnp.float32)
    return jnp.concatenate([w[:dh], z, w[dh:], z], axis=0)


def kernel(x_0, edge_index_0, batch_0, x_1, edge_index_1, batch_1,
           W1, b1, W2, b2, W3, b3, W4, b4,
           Wg0a, bg0a, Wg0b, bg0b, gamma0, beta0,
           Wg1a, bg1a, Wg1b, bg1b, gamma1, beta1,
           Wf0a, bf0a, Wf0b, bf0b, Wf1a, bf1a, Wf1b, bf1b):
    x0p = jnp.pad(x_0, ((0, NP - N0), (0, 96 - 93)))
    x1p = jnp.pad(x_1, ((0, NP - N0), (0, 48 - 43)))
    src0 = jnp.pad(edge_index_0[0], (0, EP - E0)).reshape(EP // 128, 128)
    dst0 = jnp.pad(edge_index_0[1], (0, EP - E0),
                   constant_values=N0).reshape(EP // 128, 128)
    src1 = jnp.pad(edge_index_1[0], (0, EP - E0)).reshape(EP // 128, 128)
    dst1 = jnp.pad(edge_index_1[1], (0, EP - E0),
                   constant_values=N0).reshape(EP // 128, 128)
    b0p = jnp.pad(batch_0, (0, NP - N0), constant_values=G)
    b1p_ = jnp.pad(batch_1, (0, NP - N0), constant_values=G)
    b0_2d = b0p.reshape(NP // L, L)
    b1_2d = b1p_.reshape(NP // L, L)

    W1p = _pad2(W1, 96, 96)
    W2p = _pad2(W2, 96, 960)
    W3p = _pad2(W3, 48, 48)
    W4p = _pad2(W4, 48, 448)
    b1v = _pad2(b1[None, :], 1, 96)
    b2v = _pad2(b2[None, :], 1, 960)
    b3v = _pad2(b3[None, :], 1, 48)
    b4v = _pad2(b4[None, :], 1, 448)

    wg0 = _cat_weight(Wg0a, 930, 960, 1024)
    wg1 = _cat_weight(Wg1a, 430, 448, 1024)
    wf0b = _pad2(Wf0b, 256, 128)
    wf1b = _pad2(Wf1b, 256, 128)
    bf0bv = _pad2(bf0b[None, :], 1, 128)
    bf1bv = _pad2(bf1b[None, :], 1, 128)

    g0, z0 = _branch(x0p, src0, dst0, b0_2d, b0p, W1p, b1v, W2p, b2v,
                     6, 16, 960,
                     wg0, bg0a[None, :], Wg0b, bg0b[None, :],
                     gamma0[None, :], beta0[None, :],
                     Wf0a, bf0a[None, :], wf0b, bf0bv)
    g1, z1 = _branch(x1p, src1, dst1, b1_2d, b1p_, W3p, b3v, W4p, b4v,
                     3, 16, 448,
                     wg1, bg1a[None, :], Wg1b, bg1b[None, :],
                     gamma1[None, :], beta1[None, :],
                     Wf1a, bf1a[None, :], wf1b, bf1bv)
    return (z0, g0, g1, z1)


# speedup vs baseline: 2.7868x; 1.0002x over previous
import functools

import jax
import jax.numpy as jnp
from jax import lax
from jax.experimental import pallas as pl
from jax.experimental.pallas import tpu as pltpu
from jax.experimental.pallas import tpu_sc as plsc

N0 = 50000
E0 = 800000
G = 512
NC = 2
NS = 16
L = 16
NW = NC * NS
NP = 53248
EB = 200
EP = NW * 128 * EB
GP = 768

def _mesh():
    return plsc.VectorSubcoreMesh(
        core_axis_name="c", subcore_axis_name="s",
        num_cores=NC, num_subcores=NS)


def _make_edge_agg(P, Dc):
    ZR = 128
    RZ = NP // NS // ZR
    RD = NP // NS

    @functools.partial(
        pl.kernel,
        out_type=jax.ShapeDtypeStruct((NC, P, NP, Dc), jnp.float32),
        mesh=_mesh(),
        compiler_params=pltpu.CompilerParams(use_tc_tiling_on_sc=False, needs_layout_passes=False),
        scratch_types=[
            pltpu.VMEM((EB, 128), jnp.int32),
            pltpu.VMEM((EB, 128), jnp.int32),
            pltpu.VMEM((128, Dc), jnp.float32),
            pltpu.VMEM((128, Dc), jnp.float32),
            pltpu.VMEM((128, Dc), jnp.float32),
            pltpu.VMEM((128, Dc), jnp.float32),
            pltpu.VMEM((ZR, Dc), jnp.float32),
            pltpu.VMEM_SHARED((NP, Dc), jnp.float32),
            pltpu.SemaphoreType.DMA,
            pltpu.SemaphoreType.DMA,
            pltpu.SemaphoreType.DMA,
            pltpu.SemaphoreType.DMA,
            pltpu.SemaphoreType.DMA,
            pltpu.SemaphoreType.DMA,
            pltpu.SemaphoreType.DMA,
            pltpu.SemaphoreType.DMA,
        ],
    )
    def k(*refs):
        src_hbm, dst_hbm = refs[0], refs[1]
        xparts = refs[2:2 + P]
        out = refs[2 + P]
        scr = refs[3 + P:]
        src_v, dst_v = scr[0], scr[1]
        rows = scr[2:6]
        zbuf, acc = scr[6], scr[7]
        semg = scr[8:12]
        sems = scr[12:16]

        cid = lax.axis_index("c")
        sid = lax.axis_index("s")
        wid = cid * NS + sid

        zv = jnp.zeros((L,), jnp.float32)
        for r in range(ZR):
            for c2 in range(Dc // L):
                zbuf[r, pl.ds(c2 * L, L)] = zv

        pltpu.sync_copy(src_hbm.at[pl.ds(wid * EB, EB)], src_v)
        pltpu.sync_copy(dst_hbm.at[pl.ds(wid * EB, EB)], dst_v)

        for p in range(P):
            for i in range(RZ):
                pltpu.sync_copy(zbuf, acc.at[pl.ds(sid * RD + i * ZR, ZR)])
            plsc.subcore_barrier()

            xp = xparts[p]
            dummy = xp.at[pl.ds(0, 128)]

            for b in range(4):
                pltpu.async_copy(xp.at[src_v.at[b]], rows[b], semg[b])

            def ebody(i, carry):
                for b in range(4):
                    j = 4 * i + b
                    pltpu.make_async_copy(dummy, rows[b], semg[b]).wait()
                    pltpu.async_copy(rows[b], acc.at[dst_v.at[j]], sems[b],
                                     add=True)
                for b in range(4):
                    j = 4 * i + b
                    jn = jnp.minimum(j + 4, EB - 1)
                    pltpu.make_async_copy(rows[b], acc.at[dst_v.at[j]],
                                          sems[b]).wait()
                    pltpu.async_copy(xp.at[src_v.at[jn]], rows[b], semg[b])
                return carry

            lax.fori_loop(0, EB // 4, ebody, 0)
            for b in range(4):
                pltpu.make_async_copy(dummy, rows[b], semg[b]).wait()
            plsc.subcore_barrier()

            pltpu.sync_copy(acc.at[pl.ds(sid * RD, RD)],
                            out.at[cid, p, pl.ds(sid * RD, RD)])
            plsc.subcore_barrier()

    return k


def _make_gin_mm(P, Dc, Dout, BR=1024):
    NIN = P * (NC + 1)

    def body(*refs):
        ins = refs[:NIN]
        wref, bref, oref = refs[NIN], refs[NIN + 1], refs[NIN + 2]
        acc = jnp.zeros((BR, Dout), jnp.float32)
        for p in range(P):
            wslice = wref[pl.ds(p * Dc, Dc), :]
            part = ins[p][...]
            for c in range(NC):
                part = part + ins[P + c * P + p][...]
            acc = acc + jnp.dot(part, wslice,
                                preferred_element_type=jnp.float32)
        oref[...] = jnp.maximum(acc + bref[...], 0.0)

    grid = (NP // BR,)
    chunk_spec = pl.BlockSpec((BR, Dc), lambda i: (i, 0))
    f = pl.pallas_call(
        body,
        grid=grid,
        in_specs=[chunk_spec] * NIN + [
            pl.BlockSpec((P * Dc, Dout), lambda i: (0, 0)),
            pl.BlockSpec((1, Dout), lambda i: (0, 0)),
        ],
        out_specs=pl.BlockSpec((BR, Dout), lambda i: (i, 0)),
        out_shape=jax.ShapeDtypeStruct((NP, Dout), jnp.float32),
    )
    return f


def _make_pool(Dhp):
    RT = NP // NW
    NB = RT // L
    NV = Dhp // L
    ZSR = 16
    RZ = GP // NS

    @functools.partial(
        pl.kernel,
        out_type=(
            jax.ShapeDtypeStruct((NC, G, Dhp), jnp.float32),
            jax.ShapeDtypeStruct((NC, G + 16, Dhp), jnp.float32),
            jax.ShapeDtypeStruct((NW * 8, Dhp), jnp.float32),
            jax.ShapeDtypeStruct((NW * 8, L), jnp.int32),
        ),
        mesh=_mesh(),
        compiler_params=pltpu.CompilerParams(use_tc_tiling_on_sc=False, needs_layout_passes=False),
        scratch_types=[
            pltpu.VMEM((L, Dhp), jnp.float32),
            pltpu.VMEM((NB, L), jnp.int32),
            pltpu.VMEM((1, Dhp), jnp.float32),
            pltpu.VMEM((8, Dhp), jnp.float32),
            pltpu.VMEM((8, L), jnp.int32),
            pltpu.VMEM((ZSR, Dhp), jnp.float32),
            pltpu.VMEM_SHARED((GP, Dhp), jnp.float32),
            pltpu.SemaphoreType.DMA,
        ],
    )
    def k(h_hbm, b2d_hbm, sumout, maxout, bvout, bidout,
          hblk, bidx, stage, bnd, idst, zst, sumacc, sem):
        cid = lax.axis_index("c")
        sid = lax.axis_index("s")
        wid = sid * NC + cid
        r0 = wid * RT
        lane = lax.iota(jnp.int32, L)

        zv = jnp.zeros((L,), jnp.float32)
        for r in range(ZSR):
            for c2 in range(NV):
                zst[r, pl.ds(c2 * L, L)] = zv

        pltpu.sync_copy(b2d_hbm.at[pl.ds(wid * NB, NB)], bidx)

        for i in range(3):
            pltpu.sync_copy(zst, sumacc.at[pl.ds(sid * RZ + i * ZSR, ZSR)])
        for i in range(3):
            pltpu.sync_copy(zst.at[pl.ds(0, 11)],
                            maxout.at[cid, pl.ds(sid * 33 + i * 11, 11)])
        for r in range(8):
            for c2 in range(NV):
                bnd[r, pl.ds(c2 * L, L)] = zv
        plsc.subcore_barrier()

        bv0 = bidx[0, pl.ds(0, L)]
        s_init = jnp.sum(jnp.where(lane == 0, bv0, 0))

        def flush(s_cur, flushed, m):
            def to_acc():
                for c2 in range(NV):
                    stage[0, pl.ds(c2 * L, L)] = m[c2]
                pltpu.sync_copy(stage, maxout.at[cid, pl.ds(s_cur, 1)])

            def to_slot0():
                for c2 in range(NV):
                    bnd[0, pl.ds(c2 * L, L)] = m[c2]

            lax.cond(flushed > 0, to_acc, to_slot0)

        def blk(i, carry):
            pltpu.sync_copy(h_hbm.at[pl.ds(r0 + i * L, L)], hblk)
            pltpu.sync_copy(hblk, sumacc.at[bidx.at[i]], add=True)
            bv = bidx[i, pl.ds(0, L)]

            def row(j, rcarry):
                s_cur, flushed, id0 = rcarry[0], rcarry[1], rcarry[2]
                m = list(rcarry[3:])
                s_j = jnp.sum(jnp.where(lane == j, bv, 0))
                neq = s_j != s_cur
                lax.cond(neq, lambda: flush(s_cur, flushed, m), lambda: None)
                id0 = jnp.where(neq & (flushed == 0), s_cur, id0)
                flushed = jnp.where(neq, 1, flushed)
                newm = []
                for c2 in range(NV):
                    rv = hblk[j, pl.ds(c2 * L, L)]
                    mv = jnp.where(neq, 0.0, m[c2])
                    newm.append(jnp.maximum(mv, rv))
                return (s_j, flushed, id0) + tuple(newm)

            return lax.fori_loop(0, L, row, carry)

        init = (s_init, jnp.int32(0), jnp.int32(1023)) + tuple(
            jnp.zeros((L,), jnp.float32) for _ in range(NV))
        fin = lax.fori_loop(0, NB, blk, init)
        s_cur, id0 = fin[0], fin[2]
        mfin = list(fin[3:])

        for c2 in range(NV):
            bnd[1, pl.ds(c2 * L, L)] = mfin[c2]
        pltpu.sync_copy(bnd, bvout.at[pl.ds(wid * 8, 8)])

        ids = jnp.where(lane == 0, id0,
                        jnp.where(lane == 1, s_cur, 1023))
        idst[0, pl.ds(0, L)] = ids
        pltpu.sync_copy(idst, bidout.at[pl.ds(wid * 8, 8)])

        plsc.subcore_barrier()

        DR = G // NS
        pltpu.sync_copy(sumacc.at[pl.ds(sid * DR, DR)],
                        sumout.at[cid, pl.ds(sid * DR, DR)])

    return k


def _make_final(Dhp):
    def body(bid_ref, sum_ref, max_ref, bval_ref, batch_ref,
             wa_ref, ba_ref, wb_ref, bb_ref, gam_ref, bet_ref,
             wfa_ref, bfa_ref, wfb_ref, bfb_ref,
             g_out, z_out, mxs):
        segid = lax.broadcasted_iota(jnp.int32, (G, 512), 0)

        def cbody(i, acc):
            chunk = jnp.reshape(batch_ref[pl.ds(i * 512, 512)], (1, 512))
            eq = (chunk == segid).astype(jnp.float32)
            return acc + jnp.sum(eq, axis=1, keepdims=True)

        cnt = lax.fori_loop(0, NP // 512, cbody,
                            jnp.zeros((G, 1), jnp.float32))

        mxs[...] = jnp.maximum(max_ref[0], max_ref[1])

        def mbody(w, carry):
            idv = bid_ref[w]
            valid = idv < G
            iw = jnp.where(valid, idv, 0)
            cur = mxs[pl.ds(iw, 1), :]
            row = bval_ref[pl.ds(w, 1), :]
            mxs[pl.ds(iw, 1), :] = jnp.where(valid,
                                             jnp.maximum(cur, row), cur)
            return carry

        lax.fori_loop(0, 2 * NW, mbody, 0)

        mean = (sum_ref[0] + sum_ref[1]) / jnp.maximum(cnt, 1.0)
        mx = mxs[...]

        t1 = jnp.dot(mean, wa_ref[pl.ds(0, Dhp), :],
                     preferred_element_type=jnp.float32)
        t1 = t1 + jnp.dot(mx, wa_ref[pl.ds(Dhp, Dhp), :],
                          preferred_element_type=jnp.float32)
        t1 = jnp.maximum(t1 + ba_ref[...], 0.0)
        g = jnp.dot(t1, wb_ref[...],
                    preferred_element_type=jnp.float32) + bb_ref[...]
        mu = jnp.mean(g, axis=0, keepdims=True)
        d = g - mu
        v = jnp.mean(d * d, axis=0, keepdims=True)
        gn = gam_ref[...] * d / jnp.sqrt(v + 1e-5) + bet_ref[...]
        g_out[...] = gn
        t2 = jnp.maximum(
            jnp.dot(gn, wfa_ref[...],
                    preferred_element_type=jnp.float32) + bfa_ref[...], 0.0)
        z_out[...] = jnp.dot(t2, wfb_ref[...],
                             preferred_element_type=jnp.float32) + bfb_ref[...]

    f = pl.pallas_call(
        body,
        in_specs=[
            pl.BlockSpec(memory_space=pltpu.SMEM),
        ] + [pl.BlockSpec()] * 14,
        out_specs=[pl.BlockSpec(), pl.BlockSpec()],
        out_shape=(
            jax.ShapeDtypeStruct((G, 512), jnp.float32),
            jax.ShapeDtypeStruct((G, 128), jnp.float32),
        ),
        scratch_shapes=[pltpu.VMEM((G, Dhp), jnp.float32)],
    )
    return f


_edge_agg = {}
_gin_mm = {}
_pool = {}
_final = {}


def _get_edge_agg(P, Dc):
    if (P, Dc) not in _edge_agg:
        _edge_agg[(P, Dc)] = _make_edge_agg(P, Dc)
    return _edge_agg[(P, Dc)]


def _branch(x, src2d, dst2d, batch2d, batchp, W1p, b1p, W2p, b2p,
            P, Dc, Dhp, wa, ba, wb, bb, gam, bet, wfa, bfa, wfb, bfb):
    D = P * Dc
    agg_fn = _get_edge_agg(P, Dc)
    mm1 = _gin_mm.setdefault((P, Dc, D), _make_gin_mm(P, Dc, D))
    mm2 = _gin_mm.setdefault((P, Dc, Dhp), _make_gin_mm(P, Dc, Dhp))
    pool_fn = _pool.setdefault(Dhp, _make_pool(Dhp))
    fin_fn = _final.setdefault(Dhp, _make_final(Dhp))

    xc = [x[:, p * Dc:(p + 1) * Dc] for p in range(P)]
    agg1 = agg_fn(src2d, dst2d, *xc)
    parts1 = [agg1[c, p] for c in range(NC) for p in range(P)]
    h1 = mm1(*xc, *parts1, W1p, b1p)

    h1c = [h1[:, p * Dc:(p + 1) * Dc] for p in range(P)]
    agg2 = agg_fn(src2d, dst2d, *h1c)
    parts2 = [agg2[c, p] for c in range(NC) for p in range(P)]
    h2 = mm2(*h1c, *parts2, W2p, b2p)

    sums, maxs, bval, bid = pool_fn(h2, batch2d)
    maxs = maxs[:, :G]
    bval2 = bval.reshape(NW, 8, Dhp)[:, :2].reshape(2 * NW, Dhp)
    bid2 = bid.reshape(NW, 8, L)[:, 0, :2].reshape(2 * NW)
    g, z = fin_fn(bid2, sums, maxs, bval2, batchp,
                  wa, ba, wb, bb, gam, bet, wfa, bfa, wfb, bfb)
    return g, z[:, :2]


def _pad2(w, r, c):
    return jnp.pad(w, ((0, r - w.shape[0]), (0, c - w.shape[1])))


def _cat_weight(w, dh, dhp, dout):
    z = jnp.zeros((dhp - dh, dout), jnp.float32)
    return jnp.concatenate([w[:dh], z, w[dh:], z], axis=0)


def kernel(x_0, edge_index_0, batch_0, x_1, edge_index_1, batch_1,
           W1, b1, W2, b2, W3, b3, W4, b4,
           Wg0a, bg0a, Wg0b, bg0b, gamma0, beta0,
           Wg1a, bg1a, Wg1b, bg1b, gamma1, beta1,
           Wf0a, bf0a, Wf0b, bf0b, Wf1a, bf1a, Wf1b, bf1b):
    x0p = jnp.pad(x_0, ((0, NP - N0), (0, 96 - 93)))
    x1p = jnp.pad(x_1, ((0, NP - N0), (0, 48 - 43)))
    src0 = jnp.pad(edge_index_0[0], (0, EP - E0)).reshape(EP // 128, 128)
    dst0 = jnp.pad(edge_index_0[1], (0, EP - E0),
                   constant_values=N0).reshape(EP // 128, 128)
    src1 = jnp.pad(edge_index_1[0], (0, EP - E0)).reshape(EP // 128, 128)
    dst1 = jnp.pad(edge_index_1[1], (0, EP - E0),
                   constant_values=N0).reshape(EP // 128, 128)
    b0p = jnp.pad(batch_0, (0, NP - N0), constant_values=G)
    b1p_ = jnp.pad(batch_1, (0, NP - N0), constant_values=G)
    b0_2d = b0p.reshape(NP // L, L)
    b1_2d = b1p_.reshape(NP // L, L)

    W1p = _pad2(W1, 96, 96)
    W2p = _pad2(W2, 96, 960)
    W3p = _pad2(W3, 48, 48)
    W4p = _pad2(W4, 48, 448)
    b1v = _pad2(b1[None, :], 1, 96)
    b2v = _pad2(b2[None, :], 1, 960)
    b3v = _pad2(b3[None, :], 1, 48)
    b4v = _pad2(b4[None, :], 1, 448)

    wg0 = _cat_weight(Wg0a, 930, 960, 1024)
    wg1 = _cat_weight(Wg1a, 430, 448, 1024)
    wf0b = _pad2(Wf0b, 256, 128)
    wf1b = _pad2(Wf1b, 256, 128)
    bf0bv = _pad2(bf0b[None, :], 1, 128)
    bf1bv = _pad2(bf1b[None, :], 1, 128)

    g0, z0 = _branch(x0p, src0, dst0, b0_2d, b0p, W1p, b1v, W2p, b2v,
                     6, 16, 960,
                     wg0, bg0a[None, :], Wg0b, bg0b[None, :],
                     gamma0[None, :], beta0[None, :],
                     Wf0a, bf0a[None, :], wf0b, bf0bv)
    g1, z1 = _branch(x1p, src1, dst1, b1_2d, b1p_, W3p, b3v, W4p, b4v,
                     3, 16, 448,
                     wg1, bg1a[None, :], Wg1b, bg1b[None, :],
                     gamma1[None, :], beta1[None, :],
                     Wf1a, bf1a[None, :], wf1b, bf1bv)
    return (z0, g0, g1, z1)
